# baseline (speedup 1.0000x reference)
import functools

import jax
import jax.numpy as jnp
from jax import lax
from jax.experimental import pallas as pl
from jax.experimental.pallas import tpu as pltpu
from jax.experimental.pallas import tpu_sc as plsc

_EPS = 0.001
_B = 8
_N = 512
_LANES = 16
_NGROUPS = _N // _LANES


def _task_params(k, wid_s, wid_c):
    wid = wid_s * 2 + wid_c
    if k == 0:
        return 0, 1.0, 2048, wid // 4, wid % 4
    if k == 1:
        return 1, 0.5, 1024, wid // 4, wid % 4
    if k == 2:
        return 2, 0.25, 512, wid // 8, wid % 8
    return 2, 0.25, 512, 4 + wid // 8, wid % 8


def _sc_body(v0, v1, v2, st_hbm, du_hbm, out_hbm,
             vbuf, obuf, start_v, dur_v, sidx_v, eidx_v, scale_v,
             sem_in, sem_out):
    wid_s = lax.axis_index("s")
    wid_c = lax.axis_index("c")
    vrefs = (v0, v1, v2)
    out_off = (0, 128, 256)

    lane_iota = lax.iota(jnp.int32, _LANES)
    lane_iota_hi = lane_iota + _LANES
    zero = jnp.zeros((_LANES,), jnp.float32)

    pltpu.sync_copy(st_hbm, start_v)
    pltpu.sync_copy(du_hbm, dur_v)

    for k in range(4):
        map_id, ratio, T, b, chunk = _task_params(k, wid_s, wid_c)
        vref = vrefs[map_id]
        c0 = chunk * 32
        outcol = out_off[map_id] + c0

        pltpu.make_async_copy(
            vref.at[b, :, pl.ds(c0, 32)], vbuf.at[pl.ds(0, T)], sem_in
        ).start()

        @plsc.parallel_loop(0, _NGROUPS, unroll=2)
        def _idx(g):
            cols = g * _LANES + lane_iota
            brow = jnp.full((_LANES,), b, jnp.int32)
            stv = plsc.load_gather(start_v, [brow, cols])
            duv = plsc.load_gather(dur_v, [brow, cols])
            sl = pl.ds(g * _LANES, _LANES)
            sf = stv * ratio
            s_i = sf.astype(jnp.int32)
            ef = (stv + duv + _EPS) * ratio
            et = ef.astype(jnp.int32)
            e_i = et + (ef > et.astype(jnp.float32)).astype(jnp.int32)
            s_i = jnp.minimum(s_i, T - 1)
            e_i = jnp.minimum(e_i, T - 1)
            cnt = (e_i - s_i).astype(jnp.float32)
            sidx_v[sl] = s_i
            eidx_v[sl] = e_i
            scale_v[sl] = 1.0 / jnp.maximum(cnt, 1.0)

        pltpu.make_async_copy(
            vref.at[b, :, pl.ds(c0, 32)], vbuf.at[pl.ds(0, T)], sem_in
        ).wait()

        @plsc.parallel_loop(0, T, unroll=8, carry=(zero, zero))
        def _cum(t, carry):
            acc_a, acc_b = carry
            row = jnp.full((_LANES,), t, jnp.int32)
            ra = plsc.load_gather(vbuf, [row, lane_iota])
            plsc.store_scatter(vbuf, [row, lane_iota], acc_a)
            rb = plsc.load_gather(vbuf, [row, lane_iota_hi])
            plsc.store_scatter(vbuf, [row, lane_iota_hi], acc_b)
            return acc_a + ra, acc_b + rb

        if k > 0:
            p_map, _, _, pb, p_chunk = _task_params(k - 1, wid_s, wid_c)
            p_col = out_off[p_map] + p_chunk * 32
            pltpu.make_async_copy(
                obuf, out_hbm.at[pb, :, pl.ds(p_col, 32)], sem_out
            ).wait()

        @plsc.parallel_loop(0, _N, unroll=2)
        def _seg(i):
            col = lax.rem(i, _LANES)
            off = i - col
            sl = pl.ds(off, _LANES)
            s_i = sidx_v[sl]
            e_i = eidx_v[sl]
            sc = scale_v[sl]
            n_vec = off + lane_iota
            col_vec = jnp.full((_LANES,), col, jnp.int32)
            col_hi = col_vec + _LANES
            ga = plsc.load_gather(vbuf, [e_i, col_vec]) - \
                plsc.load_gather(vbuf, [s_i, col_vec])
            plsc.store_scatter(obuf, [n_vec, col_vec], ga * sc)
            gb = plsc.load_gather(vbuf, [e_i, col_hi]) - \
                plsc.load_gather(vbuf, [s_i, col_hi])
            plsc.store_scatter(obuf, [n_vec, col_hi], gb * sc)

        pltpu.make_async_copy(
            obuf, out_hbm.at[b, :, pl.ds(outcol, 32)], sem_out
        ).start()

    l_map, _, _, lb, l_chunk = _task_params(3, wid_s, wid_c)
    l_col = out_off[l_map] + l_chunk * 32
    pltpu.make_async_copy(
        obuf, out_hbm.at[lb, :, pl.ds(l_col, 32)], sem_out
    ).wait()


@jax.jit
def kernel(value_list_0, value_list_1, value_list_2, start, duration):
    mesh = plsc.VectorSubcoreMesh(core_axis_name="c", subcore_axis_name="s")
    run = functools.partial(
        pl.kernel,
        mesh=mesh,
        out_type=jax.ShapeDtypeStruct((_B, _N, 512), jnp.float32),
        compiler_params=pltpu.CompilerParams(
            use_tc_tiling_on_sc=False, needs_layout_passes=False
        ),
        scratch_types=[
            pltpu.VMEM((2048, 2 * _LANES), jnp.float32),
            pltpu.VMEM((_N, 2 * _LANES), jnp.float32),
            pltpu.VMEM((_B, _N), jnp.float32),
            pltpu.VMEM((_B, _N), jnp.float32),
            pltpu.VMEM((_N,), jnp.int32),
            pltpu.VMEM((_N,), jnp.int32),
            pltpu.VMEM((_N,), jnp.float32),
            pltpu.SemaphoreType.DMA,
            pltpu.SemaphoreType.DMA,
        ],
    )(_sc_body)
    return run(value_list_0, value_list_1, value_list_2, start, duration)


# speedup vs baseline: 1.3347x; 1.3347x over previous
import functools

import jax
import jax.numpy as jnp
from jax import lax
from jax.experimental import pallas as pl
from jax.experimental.pallas import tpu as pltpu
from jax.experimental.pallas import tpu_sc as plsc

_EPS = 0.001
_B = 8
_N = 512
_LANES = 16
_NGROUPS = _N // _LANES


def _task_params(k, wid_s, wid_c):
    wid = wid_s * 2 + wid_c
    if k == 0:
        return 0, 1.0, 2048, wid // 4, wid % 4
    if k == 1:
        return 1, 0.5, 1024, wid // 4, wid % 4
    if k == 2:
        return 2, 0.25, 512, wid // 8, wid % 8
    return 2, 0.25, 512, 4 + wid // 8, wid % 8


def _sc_body(v0, v1, v2, st_hbm, du_hbm, out_hbm,
             vbuf_a, vbuf_b, obuf_a, obuf_b,
             start_v, dur_v, sidx_v, eidx_v, scale_v,
             sem_in, sem_out):
    wid_s = lax.axis_index("s")
    wid_c = lax.axis_index("c")
    vrefs = (v0, v1, v2)
    out_off = (0, 128, 256)

    lane_iota = lax.iota(jnp.int32, _LANES)
    zero = jnp.zeros((_LANES,), jnp.float32)

    pltpu.sync_copy(st_hbm, start_v)
    pltpu.sync_copy(du_hbm, dur_v)

    for k in range(4):
        map_id, ratio, T, b, chunk = _task_params(k, wid_s, wid_c)
        vref = vrefs[map_id]
        c0 = chunk * 32
        outcol = out_off[map_id] + c0

        cp_a = pltpu.make_async_copy(
            vref.at[b, :, pl.ds(c0, 16)], vbuf_a.at[pl.ds(0, T)], sem_in)
        cp_b = pltpu.make_async_copy(
            vref.at[b, :, pl.ds(c0 + 16, 16)], vbuf_b.at[pl.ds(0, T)],
            sem_in)
        cp_a.start()
        cp_b.start()

        @plsc.parallel_loop(0, _NGROUPS, unroll=2)
        def _idx(g):
            cols = g * _LANES + lane_iota
            brow = jnp.full((_LANES,), b, jnp.int32)
            stv = plsc.load_gather(start_v, [brow, cols])
            duv = plsc.load_gather(dur_v, [brow, cols])
            sl = pl.ds(g * _LANES, _LANES)
            sf = stv * ratio
            s_i = sf.astype(jnp.int32)
            ef = (stv + duv + _EPS) * ratio
            et = ef.astype(jnp.int32)
            e_i = et + (ef > et.astype(jnp.float32)).astype(jnp.int32)
            s_i = jnp.minimum(s_i, T - 1)
            e_i = jnp.minimum(e_i, T - 1)
            cnt = (e_i - s_i).astype(jnp.float32)
            sidx_v[sl] = s_i
            eidx_v[sl] = e_i
            scale_v[sl] = 1.0 / jnp.maximum(cnt, 1.0)

        cp_a.wait()
        cp_b.wait()

        @plsc.parallel_loop(0, T, unroll=8, carry=(zero, zero))
        def _cum(t, carry):
            acc_a, acc_b = carry
            row = jnp.full((_LANES,), t, jnp.int32)
            ra = plsc.load_gather(vbuf_a, [row, lane_iota])
            plsc.store_scatter(vbuf_a, [row, lane_iota], acc_a)
            rb = plsc.load_gather(vbuf_b, [row, lane_iota])
            plsc.store_scatter(vbuf_b, [row, lane_iota], acc_b)
            return acc_a + ra, acc_b + rb

        if k > 0:
            p_map, _, _, pb, p_chunk = _task_params(k - 1, wid_s, wid_c)
            p_col = out_off[p_map] + p_chunk * 32
            pltpu.make_async_copy(
                obuf_a, out_hbm.at[pb, :, pl.ds(p_col, 16)], sem_out
            ).wait()
            pltpu.make_async_copy(
                obuf_b, out_hbm.at[pb, :, pl.ds(p_col + 16, 16)], sem_out
            ).wait()

        @plsc.parallel_loop(0, _N, unroll=2)
        def _seg(i):
            col = lax.rem(i, _LANES)
            off = i - col
            sl = pl.ds(off, _LANES)
            s_i = sidx_v[sl]
            e_i = eidx_v[sl]
            sc = scale_v[sl]
            n_vec = off + lane_iota
            col_vec = jnp.full((_LANES,), col, jnp.int32)
            ga = plsc.load_gather(vbuf_a, [e_i, col_vec]) - \
                plsc.load_gather(vbuf_a, [s_i, col_vec])
            plsc.store_scatter(obuf_a, [n_vec, col_vec], ga * sc)
            gb = plsc.load_gather(vbuf_b, [e_i, col_vec]) - \
                plsc.load_gather(vbuf_b, [s_i, col_vec])
            plsc.store_scatter(obuf_b, [n_vec, col_vec], gb * sc)

        pltpu.make_async_copy(
            obuf_a, out_hbm.at[b, :, pl.ds(outcol, 16)], sem_out
        ).start()
        pltpu.make_async_copy(
            obuf_b, out_hbm.at[b, :, pl.ds(outcol + 16, 16)], sem_out
        ).start()

    l_map, _, _, lb, l_chunk = _task_params(3, wid_s, wid_c)
    l_col = out_off[l_map] + l_chunk * 32
    pltpu.make_async_copy(
        obuf_a, out_hbm.at[lb, :, pl.ds(l_col, 16)], sem_out
    ).wait()
    pltpu.make_async_copy(
        obuf_b, out_hbm.at[lb, :, pl.ds(l_col + 16, 16)], sem_out
    ).wait()


@jax.jit
def kernel(value_list_0, value_list_1, value_list_2, start, duration):
    mesh = plsc.VectorSubcoreMesh(core_axis_name="c", subcore_axis_name="s")
    run = functools.partial(
        pl.kernel,
        mesh=mesh,
        out_type=jax.ShapeDtypeStruct((_B, _N, 512), jnp.float32),
        compiler_params=pltpu.CompilerParams(
            use_tc_tiling_on_sc=False, needs_layout_passes=False
        ),
        scratch_types=[
            pltpu.VMEM((2048, _LANES), jnp.float32),
            pltpu.VMEM((2048, _LANES), jnp.float32),
            pltpu.VMEM((_N, _LANES), jnp.float32),
            pltpu.VMEM((_N, _LANES), jnp.float32),
            pltpu.VMEM((_B, _N), jnp.float32),
            pltpu.VMEM((_B, _N), jnp.float32),
            pltpu.VMEM((_N,), jnp.int32),
            pltpu.VMEM((_N,), jnp.int32),
            pltpu.VMEM((_N,), jnp.float32),
            pltpu.SemaphoreType.DMA,
            pltpu.SemaphoreType.DMA,
        ],
    )(_sc_body)
    return run(value_list_0, value_list_1, value_list_2, start, duration)


# speedup vs baseline: 1.3406x; 1.0044x over previous
import functools

import jax
import jax.numpy as jnp
from jax import lax
from jax.experimental import pallas as pl
from jax.experimental.pallas import tpu as pltpu
from jax.experimental.pallas import tpu_sc as plsc

_EPS = 0.001
_B = 8
_N = 512
_LANES = 16
_NGROUPS = _N // _LANES


def _task_params(k, wid_s, wid_c):
    wid = wid_s * 2 + wid_c
    if k == 0:
        return 0, 1.0, 2048, wid // 4, wid % 4
    if k == 1:
        return 1, 0.5, 1024, wid // 4, wid % 4
    if k == 2:
        return 2, 0.25, 512, wid // 8, wid % 8
    return 2, 0.25, 512, 4 + wid // 8, wid % 8


def _sc_body(v0, v1, v2, st_hbm, du_hbm, out_hbm,
             vbuf_a, obuf_a,
             start_v, dur_v, sidx_v, eidx_v, scale_v,
             sem_in, sem_out):
    wid_s = lax.axis_index("s")
    wid_c = lax.axis_index("c")
    vrefs = (v0, v1, v2)
    out_off = (0, 128, 256)

    lane_iota = lax.iota(jnp.int32, _LANES)
    lane_hi = lane_iota + _LANES
    zero = jnp.zeros((_LANES,), jnp.float32)

    pltpu.sync_copy(st_hbm, start_v)
    pltpu.sync_copy(du_hbm, dur_v)

    for k in range(4):
        map_id, ratio, T, b, chunk = _task_params(k, wid_s, wid_c)
        vref = vrefs[map_id]
        c0 = chunk * 32
        outcol = out_off[map_id] + c0

        cp_a = pltpu.make_async_copy(
            vref.at[b, :, pl.ds(c0, 32)],
            vbuf_a.at[pl.ds(0, T), pl.ds(0, 32)], sem_in)
        cp_a.start()

        @plsc.parallel_loop(0, _NGROUPS, unroll=2)
        def _idx(g):
            cols = g * _LANES + lane_iota
            brow = jnp.full((_LANES,), b, jnp.int32)
            stv = plsc.load_gather(start_v, [brow, cols])
            duv = plsc.load_gather(dur_v, [brow, cols])
            sl = pl.ds(g * _LANES, _LANES)
            sf = stv * ratio
            s_i = sf.astype(jnp.int32)
            ef = (stv + duv + _EPS) * ratio
            et = ef.astype(jnp.int32)
            e_i = et + (ef > et.astype(jnp.float32)).astype(jnp.int32)
            s_i = jnp.minimum(s_i, T - 1)
            e_i = jnp.minimum(e_i, T - 1)
            cnt = (e_i - s_i).astype(jnp.float32)
            sidx_v[sl] = s_i
            eidx_v[sl] = e_i
            scale_v[sl] = 1.0 / jnp.maximum(cnt, 1.0)

        cp_a.wait()

        @plsc.parallel_loop(0, T, unroll=8, carry=(zero, zero))
        def _cum(t, carry):
            acc_a, acc_b = carry
            row = jnp.full((_LANES,), t, jnp.int32)
            ra = plsc.load_gather(vbuf_a, [row, lane_iota])
            plsc.store_scatter(vbuf_a, [row, lane_iota], acc_a)
            rb = plsc.load_gather(vbuf_a, [row, lane_hi])
            plsc.store_scatter(vbuf_a, [row, lane_hi], acc_b)
            return acc_a + ra, acc_b + rb

        if k > 0:
            p_map, _, _, pb, p_chunk = _task_params(k - 1, wid_s, wid_c)
            p_col = out_off[p_map] + p_chunk * 32
            pltpu.make_async_copy(
                obuf_a.at[pl.ds(0, _N), pl.ds(0, 32)],
                out_hbm.at[pb, :, pl.ds(p_col, 32)], sem_out
            ).wait()

        @plsc.parallel_loop(0, _N, unroll=2)
        def _seg(i):
            col = lax.rem(i, _LANES)
            off = i - col
            sl = pl.ds(off, _LANES)
            s_i = sidx_v[sl]
            e_i = eidx_v[sl]
            sc = scale_v[sl]
            n_vec = off + lane_iota
            col_vec = jnp.full((_LANES,), col, jnp.int32)
            col_hi = col_vec + _LANES
            ga = plsc.load_gather(vbuf_a, [e_i, col_vec]) - \
                plsc.load_gather(vbuf_a, [s_i, col_vec])
            plsc.store_scatter(obuf_a, [n_vec, col_vec], ga * sc)
            gb = plsc.load_gather(vbuf_a, [e_i, col_hi]) - \
                plsc.load_gather(vbuf_a, [s_i, col_hi])
            plsc.store_scatter(obuf_a, [n_vec, col_hi], gb * sc)

        pltpu.make_async_copy(
            obuf_a.at[pl.ds(0, _N), pl.ds(0, 32)],
            out_hbm.at[b, :, pl.ds(outcol, 32)], sem_out
        ).start()

    l_map, _, _, lb, l_chunk = _task_params(3, wid_s, wid_c)
    l_col = out_off[l_map] + l_chunk * 32
    pltpu.make_async_copy(
        obuf_a.at[pl.ds(0, _N), pl.ds(0, 32)],
        out_hbm.at[lb, :, pl.ds(l_col, 32)], sem_out
    ).wait()


@jax.jit
def kernel(value_list_0, value_list_1, value_list_2, start, duration):
    mesh = plsc.VectorSubcoreMesh(core_axis_name="c", subcore_axis_name="s")
    run = functools.partial(
        pl.kernel,
        mesh=mesh,
        out_type=jax.ShapeDtypeStruct((_B, _N, 512), jnp.float32),
        compiler_params=pltpu.CompilerParams(
            use_tc_tiling_on_sc=False, needs_layout_passes=False
        ),
        scratch_types=[
            pltpu.VMEM((2048, 33), jnp.float32),
            pltpu.VMEM((_N, 33), jnp.float32),
            pltpu.VMEM((_B, _N), jnp.float32),
            pltpu.VMEM((_B, _N), jnp.float32),
            pltpu.VMEM((_N,), jnp.int32),
            pltpu.VMEM((_N,), jnp.int32),
            pltpu.VMEM((_N,), jnp.float32),
            pltpu.SemaphoreType.DMA,
            pltpu.SemaphoreType.DMA,
        ],
    )(_sc_body)
    return run(value_list_0, value_list_1, value_list_2, start, duration)
